# Initial kernel scaffold; baseline (speedup 1.0000x reference)
import jax
import jax.numpy as jnp
from jax.experimental import pallas as pl


def kernel(interpolated, rpn_boxes, params):
    raise NotImplementedError("write your pallas kernel here")


# speedup vs baseline: 8.1894x; 8.1894x over previous
import functools

import jax
import jax.numpy as jnp
from jax import lax
from jax.experimental import pallas as pl
from jax.experimental.pallas import tpu as pltpu
from jax.experimental.pallas import tpu_sc as plsc

_T = 224.0
_THR = 0.5
_N = 5000
_F = 65
_H = 64
_K = 96
_NJ = 5120
_TI = 200
_R = 40
_NW = 32
_ROWS_PW = 157
_BPW = _N * _K // _NW
_CH = 600
_DP = 128


def _iou_body(bi_ref, bjt_ref, out_ref):
    bi = bi_ref[:]
    bjt = bjt_ref[:]
    x1i, y1i, x2i, y2i = bi[:, 0:1], bi[:, 1:2], bi[:, 2:3], bi[:, 3:4]
    x1j, y1j, x2j, y2j = bjt[0:1, :], bjt[1:2, :], bjt[2:3, :], bjt[3:4, :]
    ai = (x2i - x1i) * (y2i - y1i)
    aj = (x2j - x1j) * (y2j - y1j)
    iw = jnp.maximum(jnp.minimum(x2i, x2j) - jnp.maximum(x1i, x1j), 0.0)
    ih = jnp.maximum(jnp.minimum(y2i, y2j) - jnp.maximum(y1i, y1j), 0.0)
    inter = iw * ih
    out_ref[:] = inter / (ai + aj - inter + 1e-8)


def _iou_call(boxes, boxes_t_pad):
    return pl.pallas_call(
        _iou_body,
        grid=(_N // _TI,),
        in_specs=[
            pl.BlockSpec((_TI, 4), lambda i: (i, 0)),
            pl.BlockSpec((4, _NJ), lambda i: (0, 0)),
        ],
        out_specs=pl.BlockSpec((_TI, _NJ), lambda i: (i, 0)),
        out_shape=jax.ShapeDtypeStruct((_N, _NJ), jnp.float32),
    )(boxes, boxes_t_pad)


def _compact_kernel(iou_hbm, idx_hbm, val_hbm, rowbuf, idxbuf, valbuf):
    wid = lax.axis_index("s") * 2 + lax.axis_index("c")
    r0 = wid * _ROWS_PW
    r1 = jnp.minimum(r0 + _ROWS_PW, _N)
    iota = lax.iota(jnp.int32, 16)
    zeros16 = jnp.zeros((16,), jnp.float32)
    izeros16 = jnp.zeros((16,), jnp.int32)

    def row_body(r, carry):
        pltpu.sync_copy(iou_hbm.at[pl.ds(r * _NJ, _NJ)], rowbuf)
        for t in range((_K + 16) // 16):
            valbuf[pl.ds(t * 16, 16)] = zeros16
            idxbuf[pl.ds(t * 16, 16)] = izeros16

        def chunk(c, cnt):
            v = rowbuf[pl.ds(c * 16, 16)]
            m = v > _THR
            mi = jnp.where(m, 1, 0)
            pos = cnt + plsc.cumsum(mi) - 1
            jv = iota + c * 16
            plsc.store_scatter(idxbuf, [pos], jv, mask=m)
            plsc.store_scatter(valbuf, [pos], v, mask=m)
            return cnt + jnp.sum(mi)

        lax.fori_loop(0, _NJ // 16, chunk, jnp.int32(0))
        pltpu.sync_copy(idxbuf.at[pl.ds(0, _K)], idx_hbm.at[pl.ds(r * _K, _K)])
        pltpu.sync_copy(valbuf.at[pl.ds(0, _K)], val_hbm.at[pl.ds(r * _K, _K)])
        return carry

    lax.fori_loop(r0, r1, row_body, jnp.int32(0))


def _compact_call(iou):
    mesh = plsc.VectorSubcoreMesh(core_axis_name="c", subcore_axis_name="s")
    k = functools.partial(
        pl.kernel,
        mesh=mesh,
        compiler_params=pltpu.CompilerParams(needs_layout_passes=False),
        out_type=[
            jax.ShapeDtypeStruct((_N * _K,), jnp.int32),
            jax.ShapeDtypeStruct((_N * _K,), jnp.float32),
        ],
        scratch_types=[
            pltpu.VMEM((_NJ,), jnp.float32),
            pltpu.VMEM((_K + 16,), jnp.int32),
            pltpu.VMEM((_K + 16,), jnp.float32),
        ],
    )(_compact_kernel)
    return k(iou)


def _gather_kernel(table_hbm, idx_hbm, out_hbm, idx_v, rows_v, sem):
    wid = lax.axis_index("s") * 2 + lax.axis_index("c")
    base0 = wid * _BPW

    def chunk(c, carry):
        base = base0 + c * _CH
        pltpu.sync_copy(idx_hbm.at[pl.ds(base, _CH)], idx_v)
        pltpu.async_copy(table_hbm.at[idx_v], rows_v, sem).wait()
        pltpu.sync_copy(rows_v, out_hbm.at[pl.ds(base, _CH)])
        return carry

    lax.fori_loop(0, _BPW // _CH, chunk, jnp.int32(0))


def _gather_call(table, idx_flat):
    mesh = plsc.VectorSubcoreMesh(core_axis_name="c", subcore_axis_name="s")
    k = functools.partial(
        pl.kernel,
        mesh=mesh,
        compiler_params=pltpu.CompilerParams(needs_layout_passes=False),
        out_type=jax.ShapeDtypeStruct((_N * _K, _DP), jnp.float32),
        scratch_types=[
            pltpu.VMEM((_CH,), jnp.int32),
            pltpu.VMEM((_CH, _DP), jnp.float32),
            pltpu.SemaphoreType.DMA,
        ],
    )(_gather_kernel)
    return k(table, idx_flat)


def _pq_body(x_ref, boxes_ref, wn_ref, wm_ref, g_ref, b1_ref, p_ref, q_ref):
    xv = x_ref[:]
    b = boxes_ref[:]
    cx = (b[:, 0:1] + b[:, 2:3]) * 0.5
    cy = (b[:, 1:2] + b[:, 3:4]) * 0.5
    w = b[:, 2:3] - b[:, 0:1]
    h = b[:, 3:4] - b[:, 1:2]
    geo = jnp.concatenate([cx, cy, w, h], axis=1) * (1.0 / _T)
    gg = jnp.dot(geo, g_ref[:], preferred_element_type=jnp.float32)
    pv = jnp.dot(xv, wn_ref[:], preferred_element_type=jnp.float32) + gg
    p_ref[:] = jnp.concatenate(
        [pv, jnp.zeros((_N, _DP - _H), jnp.float32)], axis=1)
    q_ref[:] = (jnp.dot(xv, wm_ref[:], preferred_element_type=jnp.float32)
                - gg + b1_ref[:])


def _pq_call(x, boxes, wn, wm, g, b1_row):
    return pl.pallas_call(
        _pq_body,
        grid=(1,),
        in_specs=[
            pl.BlockSpec((_N, _F), lambda i: (0, 0)),
            pl.BlockSpec((_N, 4), lambda i: (0, 0)),
            pl.BlockSpec((_F, _H), lambda i: (0, 0)),
            pl.BlockSpec((_F, _H), lambda i: (0, 0)),
            pl.BlockSpec((4, _H), lambda i: (0, 0)),
            pl.BlockSpec((1, _H), lambda i: (0, 0)),
        ],
        out_specs=[
            pl.BlockSpec((_N, _DP), lambda i: (0, 0)),
            pl.BlockSpec((_N, _H), lambda i: (0, 0)),
        ],
        out_shape=[
            jax.ShapeDtypeStruct((_N, _DP), jnp.float32),
            jax.ShapeDtypeStruct((_N, _H), jnp.float32),
        ],
    )(x, boxes, wn, wm, g, b1_row)


def _pair_body(gp_ref, q_ref, vf_ref, x_ref, g0_ref, w2_ref, b2_ref,
               wo_ref, bo_ref, out_ref):
    gp = gp_ref[:, 0:_H]
    q = q_ref[:]
    vf = vf_ref[:]
    qrep = jnp.broadcast_to(q[:, None, :], (_R, _K, _H)).reshape(_R * _K, _H)
    z = gp + qrep + vf * g0_ref[:]
    h1 = jnp.maximum(z, 0.0)
    y = jnp.dot(h1, w2_ref[:], preferred_element_type=jnp.float32)
    y = jnp.where(vf > _THR, y, -jnp.inf)
    pooled = jnp.max(y.reshape(_R, _K, _H), axis=1)
    pooled = jnp.maximum(pooled + b2_ref[:], 0.0)
    out_ref[:] = (x_ref[:]
                  + jnp.dot(pooled, wo_ref[:], preferred_element_type=jnp.float32)
                  + bo_ref[:])


def _pair_call(gp, q, vflat, x, g0, w2, b2_row, wo, bo_row):
    return pl.pallas_call(
        _pair_body,
        grid=(_N // _R,),
        in_specs=[
            pl.BlockSpec((_R * _K, _DP), lambda i: (i, 0)),
            pl.BlockSpec((_R, _H), lambda i: (i, 0)),
            pl.BlockSpec((_R * _K, 1), lambda i: (i, 0)),
            pl.BlockSpec((_R, _F), lambda i: (i, 0)),
            pl.BlockSpec((1, _H), lambda i: (0, 0)),
            pl.BlockSpec((_H, _H), lambda i: (0, 0)),
            pl.BlockSpec((1, _H), lambda i: (0, 0)),
            pl.BlockSpec((_H, _F), lambda i: (0, 0)),
            pl.BlockSpec((1, _F), lambda i: (0, 0)),
        ],
        out_specs=pl.BlockSpec((_R, _F), lambda i: (i, 0)),
        out_shape=jax.ShapeDtypeStruct((_N, _F), jnp.float32),
    )(gp, q, vflat, x, g0, w2, b2_row, wo, bo_row)


def _head_body(x_ref, w1_ref, b1_ref, w2_ref, b2_ref, out_ref):
    hv = jnp.maximum(
        jnp.dot(x_ref[:], w1_ref[:], preferred_element_type=jnp.float32)
        + b1_ref[:], 0.0)
    out_ref[:] = (jnp.dot(hv, w2_ref[:], preferred_element_type=jnp.float32)
                  + b2_ref[:])


def _head_call(x, w1, b1_row, w2, b2_row):
    return pl.pallas_call(
        _head_body,
        grid=(1,),
        in_specs=[
            pl.BlockSpec((_N, _F), lambda i: (0, 0)),
            pl.BlockSpec((_F, _H), lambda i: (0, 0)),
            pl.BlockSpec((1, _H), lambda i: (0, 0)),
            pl.BlockSpec((_H, 1), lambda i: (0, 0)),
            pl.BlockSpec((1, 1), lambda i: (0, 0)),
        ],
        out_specs=pl.BlockSpec((_N, 1), lambda i: (0, 0)),
        out_shape=jax.ShapeDtypeStruct((_N, 1), jnp.float32),
    )(x, w1, b1_row, w2, b2_row)


def kernel(interpolated, rpn_boxes, params):
    x = interpolated
    boxes = rpn_boxes
    boxes_t_pad = jnp.pad(boxes.T, ((0, 0), (0, _NJ - _N)))
    iou = _iou_call(boxes, boxes_t_pad)
    idx_flat, vals = _compact_call(iou.reshape(-1))
    vflat = vals.reshape(-1, 1)
    for blk in params["blocks"]:
        w1 = blk["W1"]
        wn, wm = w1[:_F], w1[_F:2 * _F]
        g0 = w1[2 * _F:2 * _F + 1]
        g = w1[2 * _F + 1:2 * _F + 5]
        p, q = _pq_call(x, boxes, wn, wm, g, blk["b1"].reshape(1, _H))
        gp = _gather_call(p, idx_flat)
        x = _pair_call(gp, q, vflat, x, g0, blk["W2"],
                       blk["b2"].reshape(1, _H), blk["Wo"],
                       blk["bo"].reshape(1, _F))
    fin = params["final"]
    return _head_call(x, fin["W1"], fin["b1"].reshape(1, _H),
                      fin["W2"], fin["b2"].reshape(1, 1))


# speedup vs baseline: 8.1937x; 1.0005x over previous
import functools

import jax
import jax.numpy as jnp
from jax import lax
from jax.experimental import pallas as pl
from jax.experimental.pallas import tpu as pltpu
from jax.experimental.pallas import tpu_sc as plsc

_T = 224.0
_THR = 0.5
_N = 5000
_F = 65
_H = 64
_K = 96
_NJ = 5120
_TI = 200
_R = 40
_NW = 32
_ROWS_PW = 157
_BPW = _N * _K // _NW
_CH = 120
_DP = 128


def _iou_body(bi_ref, bjt_ref, out_ref):
    bi = bi_ref[:]
    bjt = bjt_ref[:]
    x1i, y1i, x2i, y2i = bi[:, 0:1], bi[:, 1:2], bi[:, 2:3], bi[:, 3:4]
    x1j, y1j, x2j, y2j = bjt[0:1, :], bjt[1:2, :], bjt[2:3, :], bjt[3:4, :]
    ai = (x2i - x1i) * (y2i - y1i)
    aj = (x2j - x1j) * (y2j - y1j)
    iw = jnp.maximum(jnp.minimum(x2i, x2j) - jnp.maximum(x1i, x1j), 0.0)
    ih = jnp.maximum(jnp.minimum(y2i, y2j) - jnp.maximum(y1i, y1j), 0.0)
    inter = iw * ih
    out_ref[:] = inter / (ai + aj - inter + 1e-8)


def _iou_call(boxes, boxes_t_pad):
    return pl.pallas_call(
        _iou_body,
        grid=(_N // _TI,),
        in_specs=[
            pl.BlockSpec((_TI, 4), lambda i: (i, 0)),
            pl.BlockSpec((4, _NJ), lambda i: (0, 0)),
        ],
        out_specs=pl.BlockSpec((_TI, _NJ), lambda i: (i, 0)),
        out_shape=jax.ShapeDtypeStruct((_N, _NJ), jnp.float32),
    )(boxes, boxes_t_pad)


def _compact_kernel(iou_hbm, idx_hbm, val_hbm, rowbuf, idxbuf, valbuf):
    wid = lax.axis_index("s") * 2 + lax.axis_index("c")
    r0 = wid * _ROWS_PW
    r1 = jnp.minimum(r0 + _ROWS_PW, _N)
    iota = lax.iota(jnp.int32, 16)
    zeros16 = jnp.zeros((16,), jnp.float32)
    izeros16 = jnp.zeros((16,), jnp.int32)

    def row_body(r, carry):
        pltpu.sync_copy(iou_hbm.at[pl.ds(r * _NJ, _NJ)], rowbuf)
        for t in range((_K + 16) // 16):
            valbuf[pl.ds(t * 16, 16)] = zeros16
            idxbuf[pl.ds(t * 16, 16)] = izeros16

        def chunk(c, cnt):
            v = rowbuf[pl.ds(c * 16, 16)]
            m = v > _THR
            mi = jnp.where(m, 1, 0)
            pos = cnt + plsc.cumsum(mi) - 1
            jv = iota + c * 16
            plsc.store_scatter(idxbuf, [pos], jv, mask=m)
            plsc.store_scatter(valbuf, [pos], v, mask=m)
            return cnt + jnp.sum(mi)

        lax.fori_loop(0, _NJ // 16, chunk, jnp.int32(0))
        pltpu.sync_copy(idxbuf.at[pl.ds(0, _K)], idx_hbm.at[pl.ds(r * _K, _K)])
        pltpu.sync_copy(valbuf.at[pl.ds(0, _K)], val_hbm.at[pl.ds(r * _K, _K)])
        return carry

    lax.fori_loop(r0, r1, row_body, jnp.int32(0))


def _compact_call(iou):
    mesh = plsc.VectorSubcoreMesh(core_axis_name="c", subcore_axis_name="s")
    k = functools.partial(
        pl.kernel,
        mesh=mesh,
        compiler_params=pltpu.CompilerParams(needs_layout_passes=False),
        out_type=[
            jax.ShapeDtypeStruct((_N * _K,), jnp.int32),
            jax.ShapeDtypeStruct((_N * _K,), jnp.float32),
        ],
        scratch_types=[
            pltpu.VMEM((_NJ,), jnp.float32),
            pltpu.VMEM((_K + 16,), jnp.int32),
            pltpu.VMEM((_K + 16,), jnp.float32),
        ],
    )(_compact_kernel)
    return k(iou)


def _gather_kernel(table_hbm, idx_hbm, out_hbm, idx_v, rows_v, sem):
    wid = lax.axis_index("s") * 2 + lax.axis_index("c")
    base0 = wid * _BPW

    def chunk(c, carry):
        base = base0 + c * _CH
        pltpu.sync_copy(idx_hbm.at[pl.ds(base, _CH)], idx_v)
        pltpu.async_copy(table_hbm.at[idx_v], rows_v, sem).wait()
        pltpu.sync_copy(rows_v, out_hbm.at[pl.ds(base, _CH)])
        return carry

    lax.fori_loop(0, _BPW // _CH, chunk, jnp.int32(0))


def _gather_call(table, idx_flat):
    mesh = plsc.VectorSubcoreMesh(core_axis_name="c", subcore_axis_name="s")
    k = functools.partial(
        pl.kernel,
        mesh=mesh,
        compiler_params=pltpu.CompilerParams(needs_layout_passes=False),
        out_type=jax.ShapeDtypeStruct((_N * _K, _DP), jnp.float32),
        scratch_types=[
            pltpu.VMEM((_CH,), jnp.int32),
            pltpu.VMEM((_CH, _DP), jnp.float32),
            pltpu.SemaphoreType.DMA,
        ],
    )(_gather_kernel)
    return k(table, idx_flat)


def _pq_body(x_ref, boxes_ref, wn_ref, wm_ref, g_ref, b1_ref, p_ref, q_ref):
    xv = x_ref[:]
    b = boxes_ref[:]
    cx = (b[:, 0:1] + b[:, 2:3]) * 0.5
    cy = (b[:, 1:2] + b[:, 3:4]) * 0.5
    w = b[:, 2:3] - b[:, 0:1]
    h = b[:, 3:4] - b[:, 1:2]
    geo = jnp.concatenate([cx, cy, w, h], axis=1) * (1.0 / _T)
    gg = jnp.dot(geo, g_ref[:], preferred_element_type=jnp.float32)
    pv = jnp.dot(xv, wn_ref[:], preferred_element_type=jnp.float32) + gg
    p_ref[:] = jnp.concatenate(
        [pv, jnp.zeros((_N, _DP - _H), jnp.float32)], axis=1)
    q_ref[:] = (jnp.dot(xv, wm_ref[:], preferred_element_type=jnp.float32)
                - gg + b1_ref[:])


def _pq_call(x, boxes, wn, wm, g, b1_row):
    return pl.pallas_call(
        _pq_body,
        grid=(1,),
        in_specs=[
            pl.BlockSpec((_N, _F), lambda i: (0, 0)),
            pl.BlockSpec((_N, 4), lambda i: (0, 0)),
            pl.BlockSpec((_F, _H), lambda i: (0, 0)),
            pl.BlockSpec((_F, _H), lambda i: (0, 0)),
            pl.BlockSpec((4, _H), lambda i: (0, 0)),
            pl.BlockSpec((1, _H), lambda i: (0, 0)),
        ],
        out_specs=[
            pl.BlockSpec((_N, _DP), lambda i: (0, 0)),
            pl.BlockSpec((_N, _H), lambda i: (0, 0)),
        ],
        out_shape=[
            jax.ShapeDtypeStruct((_N, _DP), jnp.float32),
            jax.ShapeDtypeStruct((_N, _H), jnp.float32),
        ],
    )(x, boxes, wn, wm, g, b1_row)


def _pair_body(gp_ref, q_ref, vf_ref, x_ref, g0_ref, w2_ref, b2_ref,
               wo_ref, bo_ref, out_ref):
    gp = gp_ref[:, 0:_H]
    q = q_ref[:]
    vf = vf_ref[:]
    qrep = jnp.broadcast_to(q[:, None, :], (_R, _K, _H)).reshape(_R * _K, _H)
    z = gp + qrep + vf * g0_ref[:]
    h1 = jnp.maximum(z, 0.0)
    y = jnp.dot(h1, w2_ref[:], preferred_element_type=jnp.float32)
    y = jnp.where(vf > _THR, y, -jnp.inf)
    pooled = jnp.max(y.reshape(_R, _K, _H), axis=1)
    pooled = jnp.maximum(pooled + b2_ref[:], 0.0)
    out_ref[:] = (x_ref[:]
                  + jnp.dot(pooled, wo_ref[:], preferred_element_type=jnp.float32)
                  + bo_ref[:])


def _pair_call(gp, q, vflat, x, g0, w2, b2_row, wo, bo_row):
    return pl.pallas_call(
        _pair_body,
        grid=(_N // _R,),
        in_specs=[
            pl.BlockSpec((_R * _K, _DP), lambda i: (i, 0)),
            pl.BlockSpec((_R, _H), lambda i: (i, 0)),
            pl.BlockSpec((_R * _K, 1), lambda i: (i, 0)),
            pl.BlockSpec((_R, _F), lambda i: (i, 0)),
            pl.BlockSpec((1, _H), lambda i: (0, 0)),
            pl.BlockSpec((_H, _H), lambda i: (0, 0)),
            pl.BlockSpec((1, _H), lambda i: (0, 0)),
            pl.BlockSpec((_H, _F), lambda i: (0, 0)),
            pl.BlockSpec((1, _F), lambda i: (0, 0)),
        ],
        out_specs=pl.BlockSpec((_R, _F), lambda i: (i, 0)),
        out_shape=jax.ShapeDtypeStruct((_N, _F), jnp.float32),
    )(gp, q, vflat, x, g0, w2, b2_row, wo, bo_row)


def _head_body(x_ref, w1_ref, b1_ref, w2_ref, b2_ref, out_ref):
    hv = jnp.maximum(
        jnp.dot(x_ref[:], w1_ref[:], preferred_element_type=jnp.float32)
        + b1_ref[:], 0.0)
    out_ref[:] = (jnp.dot(hv, w2_ref[:], preferred_element_type=jnp.float32)
                  + b2_ref[:])


def _head_call(x, w1, b1_row, w2, b2_row):
    return pl.pallas_call(
        _head_body,
        grid=(1,),
        in_specs=[
            pl.BlockSpec((_N, _F), lambda i: (0, 0)),
            pl.BlockSpec((_F, _H), lambda i: (0, 0)),
            pl.BlockSpec((1, _H), lambda i: (0, 0)),
            pl.BlockSpec((_H, 1), lambda i: (0, 0)),
            pl.BlockSpec((1, 1), lambda i: (0, 0)),
        ],
        out_specs=pl.BlockSpec((_N, 1), lambda i: (0, 0)),
        out_shape=jax.ShapeDtypeStruct((_N, 1), jnp.float32),
    )(x, w1, b1_row, w2, b2_row)


def kernel(interpolated, rpn_boxes, params):
    x = interpolated
    boxes = rpn_boxes
    boxes_t_pad = jnp.pad(boxes.T, ((0, 0), (0, _NJ - _N)))
    iou = _iou_call(boxes, boxes_t_pad)
    idx_flat, vals = _compact_call(iou.reshape(-1))
    vflat = vals.reshape(-1, 1)
    for blk in params["blocks"]:
        w1 = blk["W1"]
        wn, wm = w1[:_F], w1[_F:2 * _F]
        g0 = w1[2 * _F:2 * _F + 1]
        g = w1[2 * _F + 1:2 * _F + 5]
        p, q = _pq_call(x, boxes, wn, wm, g, blk["b1"].reshape(1, _H))
        gp = _gather_call(p, idx_flat)
        x = _pair_call(gp, q, vflat, x, g0, blk["W2"],
                       blk["b2"].reshape(1, _H), blk["Wo"],
                       blk["bo"].reshape(1, _F))
    fin = params["final"]
    return _head_call(x, fin["W1"], fin["b1"].reshape(1, _H),
                      fin["W2"], fin["b2"].reshape(1, 1))


# speedup vs baseline: 27.5728x; 3.3651x over previous
import functools

import jax
import jax.numpy as jnp
from jax import lax
from jax.experimental import pallas as pl
from jax.experimental.pallas import tpu as pltpu
from jax.experimental.pallas import tpu_sc as plsc

_T = 224.0
_THR = 0.5
_N = 5000
_F = 65
_H = 64
_K = 128
_NJ = 5120
_TI = 160
_R = 128
_NW = 32
_ROWS_PW = 160
_NP = _NJ * _K
_CH = 8192


def _iou_body(bi_ref, bjt_ref, out_ref):
    bi = bi_ref[:]
    bjt = bjt_ref[:]
    x1i, y1i, x2i, y2i = bi[:, 0:1], bi[:, 1:2], bi[:, 2:3], bi[:, 3:4]
    x1j, y1j, x2j, y2j = bjt[0:1, :], bjt[1:2, :], bjt[2:3, :], bjt[3:4, :]
    ai = (x2i - x1i) * (y2i - y1i)
    aj = (x2j - x1j) * (y2j - y1j)
    iw = jnp.maximum(jnp.minimum(x2i, x2j) - jnp.maximum(x1i, x1j), 0.0)
    ih = jnp.maximum(jnp.minimum(y2i, y2j) - jnp.maximum(y1i, y1j), 0.0)
    inter = iw * ih
    out_ref[:] = inter / (ai + aj - inter + 1e-8)


def _iou_call(boxes_pad, boxes_t_pad):
    return pl.pallas_call(
        _iou_body,
        grid=(_NJ // _TI,),
        in_specs=[
            pl.BlockSpec((_TI, 4), lambda i: (i, 0)),
            pl.BlockSpec((4, _NJ), lambda i: (0, 0)),
        ],
        out_specs=pl.BlockSpec((_TI, _NJ), lambda i: (i, 0)),
        out_shape=jax.ShapeDtypeStruct((_NJ, _NJ), jnp.float32),
    )(boxes_pad, boxes_t_pad)


def _compact_kernel(iou_hbm, idx_hbm, val_hbm, rowbuf, idxbuf, valbuf):
    wid = lax.axis_index("s") * 2 + lax.axis_index("c")
    r0 = wid * _ROWS_PW
    r1 = r0 + _ROWS_PW
    iota = lax.iota(jnp.int32, 16)
    zeros16 = jnp.zeros((16,), jnp.float32)
    izeros16 = jnp.zeros((16,), jnp.int32)

    def row_body(r, carry):
        pltpu.sync_copy(iou_hbm.at[pl.ds(r * _NJ, _NJ)], rowbuf)
        for t in range((_K + 16) // 16):
            valbuf[pl.ds(t * 16, 16)] = zeros16
            idxbuf[pl.ds(t * 16, 16)] = izeros16

        def chunk(c, cnt):
            v = rowbuf[pl.ds(c * 16, 16)]
            m = v > _THR
            mi = jnp.where(m, 1, 0)
            pos = cnt + plsc.cumsum(mi) - 1
            jv = iota + c * 16
            plsc.store_scatter(idxbuf, [pos], jv, mask=m)
            plsc.store_scatter(valbuf, [pos], v, mask=m)
            return cnt + jnp.sum(mi)

        lax.fori_loop(0, _NJ // 16, chunk, jnp.int32(0))
        pltpu.sync_copy(idxbuf.at[pl.ds(0, _K)], idx_hbm.at[pl.ds(r * _K, _K)])
        pltpu.sync_copy(valbuf.at[pl.ds(0, _K)], val_hbm.at[pl.ds(r * _K, _K)])
        return carry

    lax.fori_loop(r0, r1, row_body, jnp.int32(0))


def _compact_call(iou_flat):
    mesh = plsc.VectorSubcoreMesh(core_axis_name="c", subcore_axis_name="s")
    k = functools.partial(
        pl.kernel,
        mesh=mesh,
        compiler_params=pltpu.CompilerParams(needs_layout_passes=False),
        out_type=[
            jax.ShapeDtypeStruct((_NP,), jnp.int32),
            jax.ShapeDtypeStruct((_NP,), jnp.float32),
        ],
        scratch_types=[
            pltpu.VMEM((_NJ,), jnp.float32),
            pltpu.VMEM((_K + 16,), jnp.int32),
            pltpu.VMEM((_K + 16,), jnp.float32),
        ],
    )(_compact_kernel)
    return k(iou_flat)


def _gather_kernel(table_hbm, idx_hbm, out_hbm, prow0, prow1, idx_v,
                   ob0, ob1):
    wid = lax.axis_index("s") * 2 + lax.axis_index("c")
    r0 = 2 * wid
    pltpu.sync_copy(table_hbm.at[pl.ds(r0 * _NJ, _NJ)], prow0)
    pltpu.sync_copy(table_hbm.at[pl.ds((r0 + 1) * _NJ, _NJ)], prow1)

    def chunk(c, carry):
        base = c * _CH
        pltpu.sync_copy(idx_hbm.at[pl.ds(base, _CH)], idx_v)

        def grp(g, carry2):
            iv = idx_v[pl.ds(g * 16, 16)]
            ob0[pl.ds(g * 16, 16)] = plsc.load_gather(prow0, [iv])
            ob1[pl.ds(g * 16, 16)] = plsc.load_gather(prow1, [iv])
            return carry2

        lax.fori_loop(0, _CH // 16, grp, jnp.int32(0))
        pltpu.sync_copy(ob0, out_hbm.at[pl.ds(r0 * _NP + base, _CH)])
        pltpu.sync_copy(ob1, out_hbm.at[pl.ds((r0 + 1) * _NP + base, _CH)])
        return carry

    lax.fori_loop(0, _NP // _CH, chunk, jnp.int32(0))


def _gather_call(table_flat, idx_flat):
    mesh = plsc.VectorSubcoreMesh(core_axis_name="c", subcore_axis_name="s")
    k = functools.partial(
        pl.kernel,
        mesh=mesh,
        compiler_params=pltpu.CompilerParams(needs_layout_passes=False),
        out_type=jax.ShapeDtypeStruct((_H * _NP,), jnp.float32),
        scratch_types=[
            pltpu.VMEM((_NJ,), jnp.float32),
            pltpu.VMEM((_NJ,), jnp.float32),
            pltpu.VMEM((_CH,), jnp.int32),
            pltpu.VMEM((_CH,), jnp.float32),
            pltpu.VMEM((_CH,), jnp.float32),
        ],
    )(_gather_kernel)
    return k(table_flat, idx_flat)


def _pq_body(xt_ref, bjt_ref, wnt_ref, wmt_ref, gt_ref, b1_ref,
             pt_ref, qt_ref):
    xt = xt_ref[:]
    bjt = bjt_ref[:]
    cx = (bjt[0:1, :] + bjt[2:3, :]) * 0.5
    cy = (bjt[1:2, :] + bjt[3:4, :]) * 0.5
    w = bjt[2:3, :] - bjt[0:1, :]
    h = bjt[3:4, :] - bjt[1:2, :]
    geo_t = jnp.concatenate([cx, cy, w, h], axis=0) * (1.0 / _T)
    gg = jnp.dot(gt_ref[:], geo_t, preferred_element_type=jnp.float32)
    pt_ref[:] = jnp.dot(wnt_ref[:], xt, preferred_element_type=jnp.float32) + gg
    qt_ref[:] = (jnp.dot(wmt_ref[:], xt, preferred_element_type=jnp.float32)
                 - gg + b1_ref[:])


def _pq_call(xt, boxes_t_pad, wnt, wmt, gt, b1_col):
    return pl.pallas_call(
        _pq_body,
        grid=(1,),
        in_specs=[
            pl.BlockSpec((_F, _NJ), lambda i: (0, 0)),
            pl.BlockSpec((4, _NJ), lambda i: (0, 0)),
            pl.BlockSpec((_H, _F), lambda i: (0, 0)),
            pl.BlockSpec((_H, _F), lambda i: (0, 0)),
            pl.BlockSpec((_H, 4), lambda i: (0, 0)),
            pl.BlockSpec((_H, 1), lambda i: (0, 0)),
        ],
        out_specs=[
            pl.BlockSpec((_H, _NJ), lambda i: (0, 0)),
            pl.BlockSpec((_H, _NJ), lambda i: (0, 0)),
        ],
        out_shape=[
            jax.ShapeDtypeStruct((_H, _NJ), jnp.float32),
            jax.ShapeDtypeStruct((_H, _NJ), jnp.float32),
        ],
    )(xt, boxes_t_pad, wnt, wmt, gt, b1_col)


def _pair_body(gpt_ref, qt_ref, vf_ref, xt_ref, g0_ref, w2t_ref, b2_ref,
               wot_ref, bo_ref, out_ref):
    gpt = gpt_ref[:]
    qt_b = qt_ref[:]
    vf = vf_ref[:]
    qrep = pltpu.repeat(qt_b, _K, axis=1)
    vfb = jnp.broadcast_to(vf, (_H, _K * _R))
    h1 = jnp.maximum(gpt + qrep + vfb * g0_ref[:], 0.0)
    y = jnp.dot(w2t_ref[:], h1, precision=jax.lax.Precision.HIGHEST,
                preferred_element_type=jnp.float32)
    y = jnp.where(vfb > _THR, y, -jnp.inf)
    half = _K * _R
    while half > _R:
        half //= 2
        y = jnp.maximum(y[:, :half], y[:, half:2 * half])
    pooled = jnp.maximum(y + b2_ref[:], 0.0)
    out_ref[:] = (xt_ref[:]
                  + jnp.dot(wot_ref[:], pooled,
                            precision=jax.lax.Precision.HIGHEST,
                            preferred_element_type=jnp.float32)
                  + bo_ref[:])


def _pair_call(gpt, qt, vf_row, xt, g0_col, w2t, b2_col, wot, bo_col):
    nt = _NJ // _R
    return pl.pallas_call(
        _pair_body,
        grid=(nt + 1,),
        in_specs=[
            pl.BlockSpec((_H, _R * _K), lambda i: (0, i % nt)),
            pl.BlockSpec((_H, _R), lambda i: (0, i % nt)),
            pl.BlockSpec((1, _R * _K), lambda i: (0, i % nt)),
            pl.BlockSpec((_F, _R), lambda i: (0, i % nt)),
            pl.BlockSpec((_H, 1), lambda i: (0, 0)),
            pl.BlockSpec((_H, _H), lambda i: (0, 0)),
            pl.BlockSpec((_H, 1), lambda i: (0, 0)),
            pl.BlockSpec((_F, _H), lambda i: (0, 0)),
            pl.BlockSpec((_F, 1), lambda i: (0, 0)),
        ],
        out_specs=pl.BlockSpec((_F, _R), lambda i: (0, i % nt)),
        out_shape=jax.ShapeDtypeStruct((_F, _NJ), jnp.float32),
    )(gpt, qt, vf_row, xt, g0_col, w2t, b2_col, wot, bo_col)


def _head_body(xt_ref, w1t_ref, b1_ref, w2t_ref, b2_ref, out_ref):
    hv = jnp.maximum(
        jnp.dot(w1t_ref[:], xt_ref[:], preferred_element_type=jnp.float32)
        + b1_ref[:], 0.0)
    out_ref[:] = (jnp.dot(w2t_ref[:], hv, preferred_element_type=jnp.float32)
                  + b2_ref[:])


def _head_call(xt, w1t, b1_col, w2t, b2_col):
    return pl.pallas_call(
        _head_body,
        grid=(1,),
        in_specs=[
            pl.BlockSpec((_F, _NJ), lambda i: (0, 0)),
            pl.BlockSpec((_H, _F), lambda i: (0, 0)),
            pl.BlockSpec((_H, 1), lambda i: (0, 0)),
            pl.BlockSpec((1, _H), lambda i: (0, 0)),
            pl.BlockSpec((1, 1), lambda i: (0, 0)),
        ],
        out_specs=pl.BlockSpec((1, _NJ), lambda i: (0, 0)),
        out_shape=jax.ShapeDtypeStruct((1, _NJ), jnp.float32),
    )(xt, w1t, b1_col, w2t, b2_col)


def kernel(interpolated, rpn_boxes, params):
    boxes = rpn_boxes
    boxes_t_pad = jnp.pad(boxes.T, ((0, 0), (0, _NJ - _N)))
    boxes_pad = jnp.pad(boxes, ((0, _NJ - _N), (0, 0)))
    xt = jnp.pad(interpolated.T, ((0, 0), (0, _NJ - _N)))
    iou = _iou_call(boxes_pad, boxes_t_pad)
    idx_flat, vals = _compact_call(iou.reshape(-1))
    idx_flat = idx_flat.reshape(_NJ // _R, _R, _K).swapaxes(1, 2).reshape(-1)
    vals = vals.reshape(_NJ // _R, _R, _K).swapaxes(1, 2).reshape(-1)
    vf_row = vals.reshape(1, _NP)
    for blk in params["blocks"]:
        w1t = blk["W1"].T
        wnt = w1t[:, :_F]
        wmt = w1t[:, _F:2 * _F]
        g0_col = w1t[:, 2 * _F:2 * _F + 1]
        gt = w1t[:, 2 * _F + 1:2 * _F + 5]
        pt, qt = _pq_call(xt, boxes_t_pad, wnt, wmt, gt,
                          blk["b1"].reshape(_H, 1))
        gpt_flat = _gather_call(pt.reshape(-1), idx_flat)
        gpt = gpt_flat.reshape(_H, _NP)
        xt = _pair_call(gpt, qt, vf_row, xt, g0_col, blk["W2"].T,
                        blk["b2"].reshape(_H, 1), blk["Wo"].T,
                        blk["bo"].reshape(_F, 1))
    fin = params["final"]
    out_t = _head_call(xt, fin["W1"].T, fin["b1"].reshape(_H, 1),
                       fin["W2"].T, fin["b2"].reshape(1, 1))
    return out_t[:, :_N].T


# speedup vs baseline: 27.5810x; 1.0003x over previous
import functools

import jax
import jax.numpy as jnp
from jax import lax
from jax.experimental import pallas as pl
from jax.experimental.pallas import tpu as pltpu
from jax.experimental.pallas import tpu_sc as plsc

_T = 224.0
_THR = 0.5
_N = 5000
_F = 65
_H = 64
_K = 128
_NJ = 5120
_TI = 160
_R = 128
_NW = 32
_ROWS_PW = 160
_NP = _NJ * _K
_CH = 8192


def _iou_body(bi_ref, bjt_ref, out_ref):
    bi = bi_ref[:]
    bjt = bjt_ref[:]
    x1i, y1i, x2i, y2i = bi[:, 0:1], bi[:, 1:2], bi[:, 2:3], bi[:, 3:4]
    x1j, y1j, x2j, y2j = bjt[0:1, :], bjt[1:2, :], bjt[2:3, :], bjt[3:4, :]
    ai = (x2i - x1i) * (y2i - y1i)
    aj = (x2j - x1j) * (y2j - y1j)
    iw = jnp.maximum(jnp.minimum(x2i, x2j) - jnp.maximum(x1i, x1j), 0.0)
    ih = jnp.maximum(jnp.minimum(y2i, y2j) - jnp.maximum(y1i, y1j), 0.0)
    inter = iw * ih
    out_ref[:] = inter / (ai + aj - inter + 1e-8)


def _iou_call(boxes_pad, boxes_t_pad):
    return pl.pallas_call(
        _iou_body,
        grid=(_NJ // _TI,),
        in_specs=[
            pl.BlockSpec((_TI, 4), lambda i: (i, 0)),
            pl.BlockSpec((4, _NJ), lambda i: (0, 0)),
        ],
        out_specs=pl.BlockSpec((_TI, _NJ), lambda i: (i, 0)),
        out_shape=jax.ShapeDtypeStruct((_NJ, _NJ), jnp.float32),
    )(boxes_pad, boxes_t_pad)


def _compact_kernel(iou_hbm, idx_hbm, val_hbm, rowbuf, idxbuf, valbuf):
    wid = lax.axis_index("s") * 2 + lax.axis_index("c")
    r0 = wid * _ROWS_PW
    r1 = r0 + _ROWS_PW
    iota = lax.iota(jnp.int32, 16)
    zeros16 = jnp.zeros((16,), jnp.float32)
    izeros16 = jnp.zeros((16,), jnp.int32)

    def row_body(r, carry):
        pltpu.sync_copy(iou_hbm.at[pl.ds(r * _NJ, _NJ)], rowbuf)
        for t in range((_K + 16) // 16):
            valbuf[pl.ds(t * 16, 16)] = zeros16
            idxbuf[pl.ds(t * 16, 16)] = izeros16

        def chunk(c, cnt):
            v = rowbuf[pl.ds(c * 16, 16)]
            m = v > _THR
            mi = jnp.where(m, 1, 0)
            pos = cnt + plsc.cumsum(mi) - 1
            jv = iota + c * 16
            plsc.store_scatter(idxbuf, [pos], jv, mask=m)
            plsc.store_scatter(valbuf, [pos], v, mask=m)
            return cnt + jnp.sum(mi)

        lax.fori_loop(0, _NJ // 16, chunk, jnp.int32(0))
        pltpu.sync_copy(idxbuf.at[pl.ds(0, _K)], idx_hbm.at[pl.ds(r * _K, _K)])
        pltpu.sync_copy(valbuf.at[pl.ds(0, _K)], val_hbm.at[pl.ds(r * _K, _K)])
        return carry

    lax.fori_loop(r0, r1, row_body, jnp.int32(0))


def _compact_call(iou_flat):
    mesh = plsc.VectorSubcoreMesh(core_axis_name="c", subcore_axis_name="s")
    k = functools.partial(
        pl.kernel,
        mesh=mesh,
        compiler_params=pltpu.CompilerParams(needs_layout_passes=False),
        out_type=[
            jax.ShapeDtypeStruct((_NP,), jnp.int32),
            jax.ShapeDtypeStruct((_NP,), jnp.float32),
        ],
        scratch_types=[
            pltpu.VMEM((_NJ,), jnp.float32),
            pltpu.VMEM((_K + 16,), jnp.int32),
            pltpu.VMEM((_K + 16,), jnp.float32),
        ],
    )(_compact_kernel)
    return k(iou_flat)


def _gather_kernel(table_hbm, idx_hbm, out_hbm, prow0, prow1, idx_v,
                   ob0, ob1):
    wid = lax.axis_index("s") * 2 + lax.axis_index("c")
    r0 = 2 * wid
    pltpu.sync_copy(table_hbm.at[pl.ds(r0 * _NJ, _NJ)], prow0)
    pltpu.sync_copy(table_hbm.at[pl.ds((r0 + 1) * _NJ, _NJ)], prow1)

    def chunk(c, carry):
        base = c * _CH
        pltpu.sync_copy(idx_hbm.at[pl.ds(base, _CH)], idx_v)

        def grp(g, carry2):
            iv = idx_v[pl.ds(g * 16, 16)]
            ob0[pl.ds(g * 16, 16)] = plsc.load_gather(prow0, [iv])
            ob1[pl.ds(g * 16, 16)] = plsc.load_gather(prow1, [iv])
            return carry2

        lax.fori_loop(0, _CH // 16, grp, jnp.int32(0))
        pltpu.sync_copy(ob0, out_hbm.at[pl.ds(r0 * _NP + base, _CH)])
        pltpu.sync_copy(ob1, out_hbm.at[pl.ds((r0 + 1) * _NP + base, _CH)])
        return carry

    lax.fori_loop(0, _NP // _CH, chunk, jnp.int32(0))


def _gather_call(table_flat, idx_flat):
    mesh = plsc.VectorSubcoreMesh(core_axis_name="c", subcore_axis_name="s")
    k = functools.partial(
        pl.kernel,
        mesh=mesh,
        compiler_params=pltpu.CompilerParams(needs_layout_passes=False),
        out_type=jax.ShapeDtypeStruct((_H * _NP,), jnp.float32),
        scratch_types=[
            pltpu.VMEM((_NJ,), jnp.float32),
            pltpu.VMEM((_NJ,), jnp.float32),
            pltpu.VMEM((_CH,), jnp.int32),
            pltpu.VMEM((_CH,), jnp.float32),
            pltpu.VMEM((_CH,), jnp.float32),
        ],
    )(_gather_kernel)
    return k(table_flat, idx_flat)


def _pq_body(xt_ref, bjt_ref, wnt_ref, wmt_ref, gt_ref, b1_ref,
             pt_ref, qt_ref):
    xt = xt_ref[:]
    bjt = bjt_ref[:]
    cx = (bjt[0:1, :] + bjt[2:3, :]) * 0.5
    cy = (bjt[1:2, :] + bjt[3:4, :]) * 0.5
    w = bjt[2:3, :] - bjt[0:1, :]
    h = bjt[3:4, :] - bjt[1:2, :]
    geo_t = jnp.concatenate([cx, cy, w, h], axis=0) * (1.0 / _T)
    hp = jax.lax.Precision.HIGHEST
    gg = jnp.dot(gt_ref[:], geo_t, precision=hp,
                 preferred_element_type=jnp.float32)
    pt_ref[:] = jnp.dot(wnt_ref[:], xt, precision=hp,
                        preferred_element_type=jnp.float32) + gg
    qt_ref[:] = (jnp.dot(wmt_ref[:], xt, precision=hp,
                         preferred_element_type=jnp.float32)
                 - gg + b1_ref[:])


def _pq_call(xt, boxes_t_pad, wnt, wmt, gt, b1_col):
    return pl.pallas_call(
        _pq_body,
        grid=(1,),
        in_specs=[
            pl.BlockSpec((_F, _NJ), lambda i: (0, 0)),
            pl.BlockSpec((4, _NJ), lambda i: (0, 0)),
            pl.BlockSpec((_H, _F), lambda i: (0, 0)),
            pl.BlockSpec((_H, _F), lambda i: (0, 0)),
            pl.BlockSpec((_H, 4), lambda i: (0, 0)),
            pl.BlockSpec((_H, 1), lambda i: (0, 0)),
        ],
        out_specs=[
            pl.BlockSpec((_H, _NJ), lambda i: (0, 0)),
            pl.BlockSpec((_H, _NJ), lambda i: (0, 0)),
        ],
        out_shape=[
            jax.ShapeDtypeStruct((_H, _NJ), jnp.float32),
            jax.ShapeDtypeStruct((_H, _NJ), jnp.float32),
        ],
    )(xt, boxes_t_pad, wnt, wmt, gt, b1_col)


def _pair_body(gpt_ref, qt_ref, vf_ref, xt_ref, g0_ref, w2t_ref, b2_ref,
               wot_ref, bo_ref, out_ref):
    gpt = gpt_ref[:]
    qt_b = qt_ref[:]
    vf = vf_ref[:]
    qrep = pltpu.repeat(qt_b, _K, axis=1)
    vfb = jnp.broadcast_to(vf, (_H, _K * _R))
    h1 = jnp.maximum(gpt + qrep + vfb * g0_ref[:], 0.0)
    y = jnp.dot(w2t_ref[:], h1, precision=jax.lax.Precision.HIGHEST,
                preferred_element_type=jnp.float32)
    y = jnp.where(vfb > _THR, y, -jnp.inf)
    half = _K * _R
    while half > _R:
        half //= 2
        y = jnp.maximum(y[:, :half], y[:, half:2 * half])
    pooled = jnp.maximum(y + b2_ref[:], 0.0)
    out_ref[:] = (xt_ref[:]
                  + jnp.dot(wot_ref[:], pooled,
                            precision=jax.lax.Precision.HIGHEST,
                            preferred_element_type=jnp.float32)
                  + bo_ref[:])


def _pair_call(gpt, qt, vf_row, xt, g0_col, w2t, b2_col, wot, bo_col):
    nt = _NJ // _R
    return pl.pallas_call(
        _pair_body,
        grid=(nt + 1,),
        in_specs=[
            pl.BlockSpec((_H, _R * _K), lambda i: (0, i % nt)),
            pl.BlockSpec((_H, _R), lambda i: (0, i % nt)),
            pl.BlockSpec((1, _R * _K), lambda i: (0, i % nt)),
            pl.BlockSpec((_F, _R), lambda i: (0, i % nt)),
            pl.BlockSpec((_H, 1), lambda i: (0, 0)),
            pl.BlockSpec((_H, _H), lambda i: (0, 0)),
            pl.BlockSpec((_H, 1), lambda i: (0, 0)),
            pl.BlockSpec((_F, _H), lambda i: (0, 0)),
            pl.BlockSpec((_F, 1), lambda i: (0, 0)),
        ],
        out_specs=pl.BlockSpec((_F, _R), lambda i: (0, i % nt)),
        out_shape=jax.ShapeDtypeStruct((_F, _NJ), jnp.float32),
    )(gpt, qt, vf_row, xt, g0_col, w2t, b2_col, wot, bo_col)


def _head_body(xt_ref, w1t_ref, b1_ref, w2t_ref, b2_ref, out_ref):
    hp = jax.lax.Precision.HIGHEST
    hv = jnp.maximum(
        jnp.dot(w1t_ref[:], xt_ref[:], precision=hp,
                preferred_element_type=jnp.float32)
        + b1_ref[:], 0.0)
    out_ref[:] = (jnp.dot(w2t_ref[:], hv, precision=hp,
                          preferred_element_type=jnp.float32)
                  + b2_ref[:])


def _head_call(xt, w1t, b1_col, w2t, b2_col):
    return pl.pallas_call(
        _head_body,
        grid=(1,),
        in_specs=[
            pl.BlockSpec((_F, _NJ), lambda i: (0, 0)),
            pl.BlockSpec((_H, _F), lambda i: (0, 0)),
            pl.BlockSpec((_H, 1), lambda i: (0, 0)),
            pl.BlockSpec((1, _H), lambda i: (0, 0)),
            pl.BlockSpec((1, 1), lambda i: (0, 0)),
        ],
        out_specs=pl.BlockSpec((1, _NJ), lambda i: (0, 0)),
        out_shape=jax.ShapeDtypeStruct((1, _NJ), jnp.float32),
    )(xt, w1t, b1_col, w2t, b2_col)


def kernel(interpolated, rpn_boxes, params):
    boxes = rpn_boxes
    boxes_t_pad = jnp.pad(boxes.T, ((0, 0), (0, _NJ - _N)))
    boxes_pad = jnp.pad(boxes, ((0, _NJ - _N), (0, 0)))
    xt = jnp.pad(interpolated.T, ((0, 0), (0, _NJ - _N)))
    iou = _iou_call(boxes_pad, boxes_t_pad)
    idx_flat, vals = _compact_call(iou.reshape(-1))
    idx_flat = idx_flat.reshape(_NJ // _R, _R, _K).swapaxes(1, 2).reshape(-1)
    vals = vals.reshape(_NJ // _R, _R, _K).swapaxes(1, 2).reshape(-1)
    vf_row = vals.reshape(1, _NP)
    for blk in params["blocks"]:
        w1t = blk["W1"].T
        wnt = w1t[:, :_F]
        wmt = w1t[:, _F:2 * _F]
        g0_col = w1t[:, 2 * _F:2 * _F + 1]
        gt = w1t[:, 2 * _F + 1:2 * _F + 5]
        pt, qt = _pq_call(xt, boxes_t_pad, wnt, wmt, gt,
                          blk["b1"].reshape(_H, 1))
        gpt_flat = _gather_call(pt.reshape(-1), idx_flat)
        gpt = gpt_flat.reshape(_H, _NP)
        xt = _pair_call(gpt, qt, vf_row, xt, g0_col, blk["W2"].T,
                        blk["b2"].reshape(_H, 1), blk["Wo"].T,
                        blk["bo"].reshape(_F, 1))
    fin = params["final"]
    out_t = _head_call(xt, fin["W1"].T, fin["b1"].reshape(_H, 1),
                       fin["W2"].T, fin["b2"].reshape(1, 1))
    return out_t[:, :_N].T
